# baseline (speedup 1.0000x reference)
import functools

import jax
import jax.numpy as jnp
from jax import lax
from jax.experimental import pallas as pl
from jax.experimental.pallas import tpu as pltpu
from jax.experimental.pallas import tpu_sc as plsc

N = 10000
NP = 10240
D_IN = 128
H = 64
OUT = 32
E = 320000

NC, NS = 2, 16
NW = NC * NS
C = 128
CHUNKS = 2560
EPAD = CHUNKS * C
CPT = CHUNKS // NW
RPT = NP // NS

_mesh = functools.partial(
    plsc.VectorSubcoreMesh, core_axis_name="c", subcore_axis_name="s")
_sc_params = pltpu.CompilerParams(use_tc_tiling_on_sc=False)


@functools.partial(
    pl.kernel,
    out_type=jax.ShapeDtypeStruct((NC, 2, NP, 8), jnp.float32),
    mesh=_mesh(),
    scratch_types=[
        pltpu.VMEM((CPT, C), jnp.int32),
        pltpu.VMEM((CPT, C), jnp.int32),
        pltpu.VMEM((C, 8), jnp.float32),
        pltpu.VMEM_SHARED((NP, 8), jnp.float32),
        pltpu.VMEM_SHARED((NP, 8), jnp.float32),
    ],
    compiler_params=_sc_params,
)
def _deg_kernel(dst1_hbm, dst2_hbm, ones_hbm, zeros_hbm, out_hbm,
                idx1_sc, idx2_sc, ones_v, d1_sh, d2_sh):
    cc = lax.axis_index("c")
    s = lax.axis_index("s")
    w = s * NC + cc
    pltpu.sync_copy(ones_hbm, ones_v)
    pltpu.sync_copy(dst1_hbm.at[pl.ds(w * CPT, CPT)], idx1_sc)
    pltpu.sync_copy(dst2_hbm.at[pl.ds(w * CPT, CPT)], idx2_sc)
    pltpu.sync_copy(zeros_hbm.at[pl.ds(s * RPT, RPT)],
                    d1_sh.at[pl.ds(s * RPT, RPT)])
    pltpu.sync_copy(zeros_hbm.at[pl.ds(s * RPT, RPT)],
                    d2_sh.at[pl.ds(s * RPT, RPT)])
    plsc.subcore_barrier()

    def body(j, _):
        pltpu.sync_copy(ones_v, d1_sh.at[idx1_sc.at[j]], add=True)
        pltpu.sync_copy(ones_v, d2_sh.at[idx2_sc.at[j]], add=True)
        return _

    lax.fori_loop(0, CPT, body, 0)
    plsc.subcore_barrier()
    pltpu.sync_copy(d1_sh.at[pl.ds(s * RPT, RPT)],
                    out_hbm.at[cc, 0, pl.ds(s * RPT, RPT)])
    pltpu.sync_copy(d2_sh.at[pl.ds(s * RPT, RPT)],
                    out_hbm.at[cc, 1, pl.ds(s * RPT, RPT)])


def _make_agg(D):
    @functools.partial(
        pl.kernel,
        out_type=jax.ShapeDtypeStruct((NC, NP, D), jnp.float32),
        mesh=_mesh(),
        scratch_types=[
            pltpu.VMEM((CPT, C), jnp.int32),
            pltpu.VMEM((CPT, C), jnp.int32),
            pltpu.VMEM((C, D), jnp.float32),
            pltpu.VMEM((C, D), jnp.float32),
            pltpu.VMEM_SHARED((NP, D), jnp.float32),
            pltpu.VMEM_SHARED((NP, D), jnp.float32),
            pltpu.SemaphoreType.DMA,
            pltpu.SemaphoreType.DMA,
            pltpu.SemaphoreType.DMA,
            pltpu.SemaphoreType.DMA,
        ],
        compiler_params=_sc_params,
    )
    def _agg(g_hbm, src_hbm, dst_hbm, out_hbm, src_sc, dst_sc,
             rows0, rows1, acc_sh, g_sh, g0, g1, s0, s1):
        cc = lax.axis_index("c")
        s = lax.axis_index("s")
        w = s * NC + cc
        pltpu.sync_copy(src_hbm.at[pl.ds(w * CPT, CPT)], src_sc)
        pltpu.sync_copy(dst_hbm.at[pl.ds(w * CPT, CPT)], dst_sc)
        pltpu.sync_copy(g_hbm.at[pl.ds(s * RPT, RPT)],
                        g_sh.at[pl.ds(s * RPT, RPT)])
        pltpu.sync_copy(g_hbm.at[pl.ds(s * RPT, RPT)],
                        acc_sh.at[pl.ds(s * RPT, RPT)])
        plsc.subcore_barrier()

        rows = (rows0, rows1)
        gsem = (g0, g1)
        ssem = (s0, s1)
        for i in range(2):
            pltpu.async_copy(g_sh.at[src_sc.at[i]], rows[i], gsem[i])

        def body(j, _):
            k = 2 * j
            for i in range(2):
                pltpu.make_async_copy(
                    g_sh.at[pl.ds(0, C)], rows[i], gsem[i]).wait()
                pltpu.async_copy(
                    rows[i], acc_sh.at[dst_sc.at[k + i]], ssem[i], add=True)
            for i in range(2):
                pltpu.make_async_copy(
                    rows[i], acc_sh.at[pl.ds(0, C)], ssem[i]).wait()
                pltpu.async_copy(
                    g_sh.at[src_sc.at[k + 2 + i]], rows[i], gsem[i])
            return _

        lax.fori_loop(0, CPT // 2 - 1, body, 0)
        for i in range(2):
            pltpu.make_async_copy(
                g_sh.at[pl.ds(0, C)], rows[i], gsem[i]).wait()
            pltpu.sync_copy(rows[i], acc_sh.at[dst_sc.at[CPT - 2 + i]],
                            add=True)

        plsc.subcore_barrier()
        pltpu.sync_copy(acc_sh.at[pl.ds(s * RPT, RPT)],
                        out_hbm.at[cc, pl.ds(s * RPT, RPT)])

    return _agg


_agg64 = _make_agg(H)
_agg32 = _make_agg(OUT)


BLK = 1024
GRID = NP // BLK

_deg_spec = pl.BlockSpec((NC, 2, BLK, 8), lambda i: (0, 0, i, 0))


def _dinv(degp_ref, layer):
    deg = degp_ref[0, layer, :, 0] + degp_ref[1, layer, :, 0] + 1.0
    return lax.rsqrt(deg)


def _mm1_body(x_ref, w1_ref, degp_ref, g1_ref):
    h = jnp.dot(x_ref[...], w1_ref[...], preferred_element_type=jnp.float32)
    g1_ref[...] = h * _dinv(degp_ref, 0)[:, None]


_mm1 = pl.pallas_call(
    _mm1_body,
    grid=(GRID,),
    in_specs=[
        pl.BlockSpec((BLK, D_IN), lambda i: (i, 0)),
        pl.BlockSpec((D_IN, H), lambda i: (0, 0)),
        _deg_spec,
    ],
    out_specs=pl.BlockSpec((BLK, H), lambda i: (i, 0)),
    out_shape=jax.ShapeDtypeStruct((NP, H), jnp.float32),
)


def _mm2_body(s1p_ref, g1_ref, degp_ref, b1_ref, w2_ref, g2_ref):
    comb = s1p_ref[0] + s1p_ref[1] - g1_ref[...]
    a = jnp.maximum(_dinv(degp_ref, 0)[:, None] * comb + b1_ref[...], 0.0)
    h2 = jnp.dot(a, w2_ref[...], preferred_element_type=jnp.float32)
    g2_ref[...] = h2 * _dinv(degp_ref, 1)[:, None]


_mm2 = pl.pallas_call(
    _mm2_body,
    grid=(GRID,),
    in_specs=[
        pl.BlockSpec((NC, BLK, H), lambda i: (0, i, 0)),
        pl.BlockSpec((BLK, H), lambda i: (i, 0)),
        _deg_spec,
        pl.BlockSpec((1, H), lambda i: (0, 0)),
        pl.BlockSpec((H, OUT), lambda i: (0, 0)),
    ],
    out_specs=pl.BlockSpec((BLK, OUT), lambda i: (i, 0)),
    out_shape=jax.ShapeDtypeStruct((NP, OUT), jnp.float32),
)


def _fin_body(s2p_ref, g2_ref, degp_ref, b2_ref, z_ref):
    comb = s2p_ref[0] + s2p_ref[1] - g2_ref[...]
    z_ref[...] = _dinv(degp_ref, 1)[:, None] * comb + b2_ref[...]


_fin = pl.pallas_call(
    _fin_body,
    grid=(GRID,),
    in_specs=[
        pl.BlockSpec((NC, BLK, OUT), lambda i: (0, i, 0)),
        pl.BlockSpec((BLK, OUT), lambda i: (i, 0)),
        _deg_spec,
        pl.BlockSpec((1, OUT), lambda i: (0, 0)),
    ],
    out_specs=pl.BlockSpec((BLK, OUT), lambda i: (i, 0)),
    out_shape=jax.ShapeDtypeStruct((NP, OUT), jnp.float32),
)


def _pad2d(v):
    fill = jnp.full((EPAD - E,), NP - 1, jnp.int32)
    return jnp.concatenate([v, fill]).reshape(CHUNKS, C)


@jax.jit
def _run(x, edge_index1, edge_index2, W1, b1, W2, b2):
    xp = jnp.zeros((NP, D_IN), jnp.float32).at[:N].set(x)
    src1, dst1 = _pad2d(edge_index1[0]), _pad2d(edge_index1[1])
    src2, dst2 = _pad2d(edge_index2[0]), _pad2d(edge_index2[1])
    ones = jnp.ones((C, 8), jnp.float32)
    zeros = jnp.zeros((NP, 8), jnp.float32)

    degp = _deg_kernel(dst1, dst2, ones, zeros)
    g1 = _mm1(xp, W1, degp)
    s1p = _agg64(g1, src1, dst1)
    g2 = _mm2(s1p, g1, degp, b1.reshape(1, H), W2)
    s2p = _agg32(g2, src2, dst2)
    z = _fin(s2p, g2, degp, b2.reshape(1, OUT))
    return z[:N]


def kernel(x, edge_index1, edge_index2, W1, b1, W2, b2):
    return _run(x, edge_index1, edge_index2, W1, b1, W2, b2)


# speedup vs baseline: 1.0410x; 1.0410x over previous
import functools

import jax
import jax.numpy as jnp
from jax import lax
from jax.experimental import pallas as pl
from jax.experimental.pallas import tpu as pltpu
from jax.experimental.pallas import tpu_sc as plsc

N = 10000
NP = 10240
D_IN = 128
H = 64
OUT = 32
E = 320000

NC, NS = 2, 16
NW = NC * NS
C = 128
CHUNKS = 2560
EPAD = CHUNKS * C
CPT = CHUNKS // NW
RPT = NP // NS

_mesh = functools.partial(
    plsc.VectorSubcoreMesh, core_axis_name="c", subcore_axis_name="s")
_sc_params = pltpu.CompilerParams(use_tc_tiling_on_sc=False)


@functools.partial(
    pl.kernel,
    out_type=jax.ShapeDtypeStruct((NC, 2, NP, 8), jnp.float32),
    mesh=_mesh(),
    scratch_types=[
        pltpu.VMEM((CPT, C), jnp.int32),
        pltpu.VMEM((CPT, C), jnp.int32),
        pltpu.VMEM((C, 8), jnp.float32),
        pltpu.VMEM_SHARED((NP, 8), jnp.float32),
        pltpu.VMEM_SHARED((NP, 8), jnp.float32),
    ],
    compiler_params=_sc_params,
)
def _deg_kernel(dst1_hbm, dst2_hbm, ones_hbm, zeros_hbm, out_hbm,
                idx1_sc, idx2_sc, ones_v, d1_sh, d2_sh):
    cc = lax.axis_index("c")
    s = lax.axis_index("s")
    w = s * NC + cc
    pltpu.sync_copy(ones_hbm, ones_v)
    pltpu.sync_copy(dst1_hbm.at[pl.ds(w * CPT, CPT)], idx1_sc)
    pltpu.sync_copy(dst2_hbm.at[pl.ds(w * CPT, CPT)], idx2_sc)
    pltpu.sync_copy(zeros_hbm.at[pl.ds(s * RPT, RPT)],
                    d1_sh.at[pl.ds(s * RPT, RPT)])
    pltpu.sync_copy(zeros_hbm.at[pl.ds(s * RPT, RPT)],
                    d2_sh.at[pl.ds(s * RPT, RPT)])
    plsc.subcore_barrier()

    def body(j, _):
        pltpu.sync_copy(ones_v, d1_sh.at[idx1_sc.at[j]], add=True)
        pltpu.sync_copy(ones_v, d2_sh.at[idx2_sc.at[j]], add=True)
        return _

    lax.fori_loop(0, CPT, body, 0)
    plsc.subcore_barrier()
    pltpu.sync_copy(d1_sh.at[pl.ds(s * RPT, RPT)],
                    out_hbm.at[cc, 0, pl.ds(s * RPT, RPT)])
    pltpu.sync_copy(d2_sh.at[pl.ds(s * RPT, RPT)],
                    out_hbm.at[cc, 1, pl.ds(s * RPT, RPT)])


def _make_agg(D):
    @functools.partial(
        pl.kernel,
        out_type=jax.ShapeDtypeStruct((NC, NP, D), jnp.float32),
        mesh=_mesh(),
        scratch_types=[
            pltpu.VMEM((CPT, C), jnp.int32),
            pltpu.VMEM((CPT, C), jnp.int32),
            pltpu.VMEM((C, D), jnp.float32),
            pltpu.VMEM((C, D), jnp.float32),
            pltpu.VMEM_SHARED((NP, D), jnp.float32),
            pltpu.VMEM_SHARED((NP, D), jnp.float32),
            pltpu.SemaphoreType.DMA,
            pltpu.SemaphoreType.DMA,
        ],
        compiler_params=_sc_params,
    )
    def _agg(g_hbm, src_hbm, dst_hbm, out_hbm, src_sc, dst_sc,
             rows0, rows1, acc_sh, g_sh, sem0, sem1):
        cc = lax.axis_index("c")
        s = lax.axis_index("s")
        w = s * NC + cc
        pltpu.sync_copy(src_hbm.at[pl.ds(w * CPT, CPT)], src_sc)
        pltpu.sync_copy(dst_hbm.at[pl.ds(w * CPT, CPT)], dst_sc)
        pltpu.sync_copy(g_hbm.at[pl.ds(s * RPT, RPT)],
                        g_sh.at[pl.ds(s * RPT, RPT)])
        pltpu.sync_copy(g_hbm.at[pl.ds(s * RPT, RPT)],
                        acc_sh.at[pl.ds(s * RPT, RPT)])
        plsc.subcore_barrier()

        pltpu.async_copy(g_sh.at[src_sc.at[0]], rows0, sem0)
        pltpu.async_copy(g_sh.at[src_sc.at[1]], rows1, sem1)

        def body(j, _):
            k = 2 * j
            pltpu.make_async_copy(g_sh.at[pl.ds(0, C)], rows0, sem0).wait()
            pltpu.sync_copy(rows0, acc_sh.at[dst_sc.at[k]], add=True)
            pltpu.async_copy(g_sh.at[src_sc.at[k + 2]], rows0, sem0)
            pltpu.make_async_copy(g_sh.at[pl.ds(0, C)], rows1, sem1).wait()
            pltpu.sync_copy(rows1, acc_sh.at[dst_sc.at[k + 1]], add=True)
            pltpu.async_copy(g_sh.at[src_sc.at[k + 3]], rows1, sem1)
            return _

        lax.fori_loop(0, CPT // 2 - 1, body, 0)
        pltpu.make_async_copy(g_sh.at[pl.ds(0, C)], rows0, sem0).wait()
        pltpu.sync_copy(rows0, acc_sh.at[dst_sc.at[CPT - 2]], add=True)
        pltpu.make_async_copy(g_sh.at[pl.ds(0, C)], rows1, sem1).wait()
        pltpu.sync_copy(rows1, acc_sh.at[dst_sc.at[CPT - 1]], add=True)

        plsc.subcore_barrier()
        pltpu.sync_copy(acc_sh.at[pl.ds(s * RPT, RPT)],
                        out_hbm.at[cc, pl.ds(s * RPT, RPT)])

    return _agg


_agg64 = _make_agg(H)
_agg32 = _make_agg(OUT)


BLK = 1024
GRID = NP // BLK

_deg_spec = pl.BlockSpec((NC, 2, BLK, 8), lambda i: (0, 0, i, 0))


def _dinv(degp_ref, layer):
    deg = degp_ref[0, layer, :, 0] + degp_ref[1, layer, :, 0] + 1.0
    return lax.rsqrt(deg)


def _mm1_body(x_ref, w1_ref, degp_ref, g1_ref):
    h = jnp.dot(x_ref[...], w1_ref[...], preferred_element_type=jnp.float32)
    g1_ref[...] = h * _dinv(degp_ref, 0)[:, None]


_mm1 = pl.pallas_call(
    _mm1_body,
    grid=(GRID,),
    in_specs=[
        pl.BlockSpec((BLK, D_IN), lambda i: (i, 0)),
        pl.BlockSpec((D_IN, H), lambda i: (0, 0)),
        _deg_spec,
    ],
    out_specs=pl.BlockSpec((BLK, H), lambda i: (i, 0)),
    out_shape=jax.ShapeDtypeStruct((NP, H), jnp.float32),
)


def _mm2_body(s1p_ref, g1_ref, degp_ref, b1_ref, w2_ref, g2_ref):
    comb = s1p_ref[0] + s1p_ref[1] - g1_ref[...]
    a = jnp.maximum(_dinv(degp_ref, 0)[:, None] * comb + b1_ref[...], 0.0)
    h2 = jnp.dot(a, w2_ref[...], preferred_element_type=jnp.float32)
    g2_ref[...] = h2 * _dinv(degp_ref, 1)[:, None]


_mm2 = pl.pallas_call(
    _mm2_body,
    grid=(GRID,),
    in_specs=[
        pl.BlockSpec((NC, BLK, H), lambda i: (0, i, 0)),
        pl.BlockSpec((BLK, H), lambda i: (i, 0)),
        _deg_spec,
        pl.BlockSpec((1, H), lambda i: (0, 0)),
        pl.BlockSpec((H, OUT), lambda i: (0, 0)),
    ],
    out_specs=pl.BlockSpec((BLK, OUT), lambda i: (i, 0)),
    out_shape=jax.ShapeDtypeStruct((NP, OUT), jnp.float32),
)


def _fin_body(s2p_ref, g2_ref, degp_ref, b2_ref, z_ref):
    comb = s2p_ref[0] + s2p_ref[1] - g2_ref[...]
    z_ref[...] = _dinv(degp_ref, 1)[:, None] * comb + b2_ref[...]


_fin = pl.pallas_call(
    _fin_body,
    grid=(GRID,),
    in_specs=[
        pl.BlockSpec((NC, BLK, OUT), lambda i: (0, i, 0)),
        pl.BlockSpec((BLK, OUT), lambda i: (i, 0)),
        _deg_spec,
        pl.BlockSpec((1, OUT), lambda i: (0, 0)),
    ],
    out_specs=pl.BlockSpec((BLK, OUT), lambda i: (i, 0)),
    out_shape=jax.ShapeDtypeStruct((NP, OUT), jnp.float32),
)


def _pad2d(v):
    fill = jnp.full((EPAD - E,), NP - 1, jnp.int32)
    return jnp.concatenate([v, fill]).reshape(CHUNKS, C)


@jax.jit
def _run(x, edge_index1, edge_index2, W1, b1, W2, b2):
    xp = jnp.zeros((NP, D_IN), jnp.float32).at[:N].set(x)
    src1, dst1 = _pad2d(edge_index1[0]), _pad2d(edge_index1[1])
    src2, dst2 = _pad2d(edge_index2[0]), _pad2d(edge_index2[1])
    ones = jnp.ones((C, 8), jnp.float32)
    zeros = jnp.zeros((NP, 8), jnp.float32)

    degp = _deg_kernel(dst1, dst2, ones, zeros)
    g1 = _mm1(xp, W1, degp)
    s1p = _agg64(g1, src1, dst1)
    g2 = _mm2(s1p, g1, degp, b1.reshape(1, H), W2)
    s2p = _agg32(g2, src2, dst2)
    z = _fin(s2p, g2, degp, b2.reshape(1, OUT))
    return z[:N]


def kernel(x, edge_index1, edge_index2, W1, b1, W2, b2):
    return _run(x, edge_index1, edge_index2, W1, b1, W2, b2)
